# baseline (speedup 1.0000x reference)
import functools

import jax
import jax.numpy as jnp
from jax import lax
from jax.experimental import pallas as pl
from jax.experimental.pallas import tpu as pltpu
from jax.experimental.pallas import tpu_sc as plsc

_D = 64
_S = 50
_NB = 16384 // 128
_NW = 32
_BLK2 = 256 * _S

_mesh = plsc.VectorSubcoreMesh(core_axis_name="c", subcore_axis_name="s")

_KIN = 256
_NCB = 999936 // _KIN
_CB_PER_W = 122


@functools.partial(
    pl.kernel,
    mesh=_mesh,
    out_type=jax.ShapeDtypeStruct((500000, 128), jnp.float32),
    scratch_types=[
        pltpu.VMEM((_D, _KIN), jnp.float32),
        pltpu.VMEM((_D, _KIN), jnp.float32),
        pltpu.VMEM((128, 128), jnp.float32),
        pltpu.VMEM((128, 128), jnp.float32),
        pltpu.VMEM((32, 128), jnp.float32),
        pltpu.SemaphoreType.DMA,
        pltpu.SemaphoreType.DMA,
        pltpu.SemaphoreType.DMA,
        pltpu.SemaphoreType.DMA,
    ],
    compiler_params=pltpu.CompilerParams(
        use_tc_tiling_on_sc=True, needs_layout_passes=False),
)
def _convert_kernel(embt_hbm, tail_hbm, out_hbm, in0, in1, ot0, ot1, tl,
                    si0, si1, so0, so1):
    wid = lax.axis_index("s") * 2 + lax.axis_index("c")
    in_v = (in0, in1)
    out_v = (ot0, ot1)
    sem_i = (si0, si1)
    sem_o = (so0, so1)

    lane = lax.iota(jnp.int32, 16)
    rows_k = [lane + (16 * k) % 64 for k in range(8)]

    def start_in(c, b):
        pltpu.async_copy(
            embt_hbm.at[:, pl.ds(c * _KIN, _KIN)], in_v[b], sem_i[b])

    def wait_in(b):
        pltpu.make_async_copy(
            embt_hbm.at[:, pl.ds(0, _KIN)], in_v[b], sem_i[b]).wait()

    def transpose(b):
        @plsc.parallel_loop(0, 128, unroll=2)
        def _(r):
            for k in range(8):
                colv = jnp.full((16,), k // 4, jnp.int32) + 2 * r
                v = plsc.load_gather(in_v[b], [rows_k[k], colv])
                out_v[b][r, pl.ds(16 * k, 16)] = v

    def start_out(c, b):
        pltpu.async_copy(out_v[b], out_hbm.at[pl.ds(c * 128, 128)], sem_o[b])

    def wait_out(b):
        pltpu.make_async_copy(
            out_v[b], out_hbm.at[pl.ds(0, 128)], sem_o[b]).wait()

    lo = wid * _CB_PER_W

    start_in(lo, 0)
    start_in(lo + 1, 1)

    def body(i, carry):
        c = lo + 2 * i

        def sub(cc, b):
            wait_in(b)

            @pl.when(i > 0)
            def _():
                wait_out(b)

            transpose(b)
            start_out(cc, b)

            @pl.when(i < _CB_PER_W // 2 - 1)
            def _():
                start_in(cc + 2, b)

        sub(c, 0)
        sub(c + 1, 1)
        return carry

    lax.fori_loop(0, _CB_PER_W // 2, body, 0)
    wait_out(0)
    wait_out(1)

    @pl.when(wid < 2)
    def _():
        c = _NCB - 2 + wid
        start_in(c, 0)
        wait_in(0)
        transpose(0)
        start_out(c, 0)
        wait_out(0)

    @pl.when(wid == 31)
    def _():
        pltpu.sync_copy(tail_hbm, tl)
        pltpu.sync_copy(tl, out_hbm.at[pl.ds(499968, 32)])


@functools.partial(
    pl.kernel,
    mesh=_mesh,
    out_type=jax.ShapeDtypeStruct((_S, 8, _NB, 8, 128), jnp.float32),
    scratch_types=[
        pltpu.VMEM((512 * _S,), jnp.int32),
        pltpu.VMEM((512,), jnp.int32),
        pltpu.VMEM((512, _D), jnp.float32),
        pltpu.VMEM((512, _D), jnp.float32),
        pltpu.VMEM((8, 4, 8, 128), jnp.float32),
        pltpu.SemaphoreType.DMA,
        pltpu.SemaphoreType.DMA,
        pltpu.SemaphoreType.DMA,
    ],
    compiler_params=pltpu.CompilerParams(
        use_tc_tiling_on_sc=False, needs_layout_passes=False),
)
def _gather_kernel(idx_hbm, table_hbm, out_hbm, blk_v, slab_idx, rows0, rows1,
                   swz, sg0, sg1, ss):
    wid = lax.axis_index("s") * 2 + lax.axis_index("c")
    rows_v = (rows0, rows1)
    sem_g = (sg0, sg1)

    lane = lax.iota(jnp.int32, 16)
    lane50 = lane * 50
    row_ids = [lane + 16 * m for m in range(32)]

    def extract_idx(s):
        base = lane50 + jnp.full((16,), s, jnp.int32)
        for k in range(32):
            v = plsc.load_gather(blk_v, [base + (k * 16 * 50)])
            slab_idx[pl.ds(k * 16, 16)] = v

    def start_gather(b):
        pltpu.async_copy(table_hbm.at[slab_idx], rows_v[b], sem_g[b])

    def wait_gather(b):
        pltpu.make_async_copy(
            table_hbm.at[pl.ds(0, 512)], rows_v[b], sem_g[b]).wait()

    def transpose(b):
        @plsc.parallel_loop(0, 8, unroll=2)
        def _(dt):
            for btd in range(4):
                for di in range(8):
                    col = jnp.full((16,), di, jnp.int32) + dt * 8
                    vs = [plsc.load_gather(
                              rows_v[b], [row_ids[btd * 8 + k], col])
                          for k in range(8)]
                    for k in range(8):
                        swz[dt, btd, di, pl.ds(k * 16, 16)] = vs[k]

    def start_store(s):
        pltpu.async_copy(swz, out_hbm.at[s, :, pl.ds(wid * 4, 4)], ss)

    def wait_store():
        pltpu.make_async_copy(swz, out_hbm.at[0, :, pl.ds(0, 4)], ss).wait()

    pltpu.sync_copy(idx_hbm.at[pl.ds(wid * 512 * _S, 512 * _S)], blk_v)
    extract_idx(0)
    start_gather(0)

    def body(i, c):
        s = 2 * i
        wait_gather(0)
        extract_idx(s + 1)
        start_gather(1)

        @pl.when(i > 0)
        def _():
            wait_store()

        transpose(0)
        start_store(s)

        wait_gather(1)

        @pl.when(i < _S // 2 - 1)
        def _():
            extract_idx(s + 2)
            start_gather(0)

        wait_store()
        transpose(1)
        start_store(s + 1)
        return c

    lax.fori_loop(0, _S // 2, body, 0)
    wait_store()


def kernel(token_ids, emb):
    idx = token_ids.reshape(-1).astype(jnp.int32)
    out5 = _gather_kernel(idx, emb)
    return out5.transpose((2, 4, 0, 1, 3)).reshape(16384, _S, _D)


# speedup vs baseline: 1.3309x; 1.3309x over previous
import functools

import jax
import jax.numpy as jnp
from jax import lax
from jax.experimental import pallas as pl
from jax.experimental.pallas import tpu as pltpu
from jax.experimental.pallas import tpu_sc as plsc

_D = 64
_B = 16384 * 50
_NW = 32
_PER_W = _B // _NW
_CHUNK = 800
_NCHUNK = _PER_W // _CHUNK

_mesh = plsc.VectorSubcoreMesh(core_axis_name="c", subcore_axis_name="s")


@functools.partial(
    pl.kernel,
    mesh=_mesh,
    out_type=jax.ShapeDtypeStruct((_B, _D), jnp.float32),
    scratch_types=[
        pltpu.VMEM((_CHUNK,), jnp.int32),
        pltpu.VMEM((_CHUNK,), jnp.int32),
        pltpu.VMEM((_CHUNK, _D), jnp.float32),
        pltpu.VMEM((_CHUNK, _D), jnp.float32),
        pltpu.SemaphoreType.DMA,
        pltpu.SemaphoreType.DMA,
        pltpu.SemaphoreType.DMA,
        pltpu.SemaphoreType.DMA,
        pltpu.SemaphoreType.DMA,
        pltpu.SemaphoreType.DMA,
    ],
    compiler_params=pltpu.CompilerParams(use_tc_tiling_on_sc=False),
)
def _gather_kernel(idx_hbm, table_hbm, out_hbm, idx0, idx1, rows0, rows1,
                   si0, si1, sg0, sg1, ss0, ss1):
    wid = lax.axis_index("s") * 2 + lax.axis_index("c")
    base = wid * _PER_W
    idx_v = (idx0, idx1)
    rows_v = (rows0, rows1)
    sem_i = (si0, si1)
    sem_g = (sg0, sg1)
    sem_s = (ss0, ss1)

    def start_idx(g, b):
        off = base + jnp.minimum(g, _NCHUNK - 1) * _CHUNK
        pltpu.async_copy(idx_hbm.at[pl.ds(off, _CHUNK)], idx_v[b], sem_i[b])

    def wait_idx(b):
        pltpu.make_async_copy(
            idx_hbm.at[pl.ds(base, _CHUNK)], idx_v[b], sem_i[b]).wait()

    def start_gather(b):
        pltpu.async_copy(table_hbm.at[idx_v[b]], rows_v[b], sem_g[b])

    def wait_gather(b):
        pltpu.make_async_copy(
            table_hbm.at[pl.ds(0, _CHUNK)], rows_v[b], sem_g[b]).wait()

    def start_store(g, b):
        off = base + g * _CHUNK
        pltpu.async_copy(rows_v[b], out_hbm.at[pl.ds(off, _CHUNK)], sem_s[b])

    def wait_store(b):
        pltpu.make_async_copy(
            rows_v[b], out_hbm.at[pl.ds(base, _CHUNK)], sem_s[b]).wait()

    pltpu.sync_copy(idx_hbm.at[pl.ds(base, _CHUNK)], idx0)
    start_gather(0)
    start_idx(1, 1)
    wait_gather(0)
    start_store(0, 0)
    wait_idx(1)
    start_gather(1)
    start_idx(2, 0)

    def chunk_body(g, b):
        nb = 1 - b
        wait_gather(b)
        start_store(g, b)
        wait_idx(nb)
        wait_store(nb)
        start_gather(nb)
        start_idx(g + 2, b)

    def body(i, carry):
        g = 1 + 2 * i
        chunk_body(g, 1)
        chunk_body(g + 1, 0)
        return carry

    lax.fori_loop(0, (_NCHUNK - 2) // 2, body, 0)

    last = _NCHUNK - 1
    b = last % 2
    nb = 1 - b
    wait_gather(b)
    start_store(last, b)
    wait_idx(nb)
    wait_store(nb)
    wait_store(b)


def kernel(token_ids, emb):
    idx = token_ids.reshape(-1).astype(jnp.int32)
    out = _gather_kernel(idx, emb)
    return out.reshape(token_ids.shape + (_D,))
